# baseline (speedup 1.0000x reference)
import jax
import jax.numpy as jnp
from jax import lax
from jax.experimental import pallas as pl
from jax.experimental.pallas import tpu as pltpu
from jax.experimental.pallas import tpu_sc as plsc

BATCH = 16384
D = 8
NUM_CORES = 2
NUM_SUBCORES = 16
NW = NUM_CORES * NUM_SUBCORES
BPW = BATCH // NW
L = 16
NCHUNK = BPW // L


def _sc_gather_body(users_hbm, tracks_hbm, utab_hbm, ttab_hbm,
                    x_out, uidx_v, tidx_v, uaddr_v, taddr_v,
                    urows_v, trows_v, sem):
    c = lax.axis_index("c")
    s = lax.axis_index("s")
    wid = s * NUM_CORES + c
    base = wid * BPW

    cu0 = pltpu.async_copy(users_hbm.at[pl.ds(base, BPW)], uidx_v, sem)
    ct0 = pltpu.async_copy(tracks_hbm.at[pl.ds(base, BPW)], tidx_v, sem)
    cu0.wait()
    ct0.wait()

    def build_addrs(idx_v, addr_v):
        for ch in range(NCHUNK):
            v = idx_v[pl.ds(ch * L, L)]
            a0 = (lax.shift_right_logical(v, 7) * 1024) + (v & 127)
            for k in range(D):
                addr_v[pl.ds(k * BPW + ch * L, L)] = a0 + k * 128

    build_addrs(uidx_v, uaddr_v)
    cu = pltpu.async_copy(utab_hbm.at[uaddr_v], urows_v, sem)
    build_addrs(tidx_v, taddr_v)
    ct = pltpu.async_copy(ttab_hbm.at[taddr_v], trows_v, sem)
    cu.wait()
    ct.wait()
    for k in range(D):
        pltpu.sync_copy(urows_v.at[pl.ds(k * BPW, BPW)],
                        x_out.at[k, pl.ds(base, BPW)])
        pltpu.sync_copy(trows_v.at[pl.ds(k * BPW, BPW)],
                        x_out.at[D + k, pl.ds(base, BPW)])


_sc_gather = pl.kernel(
    _sc_gather_body,
    out_type=jax.ShapeDtypeStruct((2 * D, BATCH), jnp.float32),
    mesh=plsc.VectorSubcoreMesh(core_axis_name="c", subcore_axis_name="s"),
    scratch_types=[
        pltpu.VMEM((BPW,), jnp.int32),
        pltpu.VMEM((BPW,), jnp.int32),
        pltpu.VMEM((D * BPW,), jnp.int32),
        pltpu.VMEM((D * BPW,), jnp.int32),
        pltpu.VMEM((D * BPW,), jnp.float32),
        pltpu.VMEM((D * BPW,), jnp.float32),
        pltpu.SemaphoreType.DMA,
    ],
    compiler_params=pltpu.CompilerParams(use_tc_tiling_on_sc=False),
)


def _mlp_body(x_ref, w1t_ref, b1_ref, w2t_ref, b2_ref,
              w3t_ref, b3_ref, o_ref):
    h = jnp.dot(w1t_ref[...], x_ref[...], preferred_element_type=jnp.float32)
    h = jnp.maximum(h + b1_ref[...], 0.0)
    h = jnp.maximum(
        jnp.dot(w2t_ref[...], h, preferred_element_type=jnp.float32)
        + b2_ref[...], 0.0)
    o = jnp.dot(w3t_ref[...], h, preferred_element_type=jnp.float32) + b3_ref[...]
    o_ref[...] = jax.nn.sigmoid(o)


def _mlp(xT, W1T, b1c, W2T, b2c, W3T, b3c, block=2048):
    grid = BATCH // block
    full = lambda shape: pl.BlockSpec(shape, lambda i: (0, 0))
    return pl.pallas_call(
        _mlp_body,
        grid=(grid,),
        in_specs=[
            pl.BlockSpec((2 * D, block), lambda i: (0, i)),
            full((64, 2 * D)),
            full((64, 1)),
            full((32, 64)),
            full((32, 1)),
            full((1, 32)),
            full((1, 1)),
        ],
        out_specs=pl.BlockSpec((1, block), lambda i: (0, i)),
        out_shape=jax.ShapeDtypeStruct((1, BATCH), jnp.float32),
    )(xT, W1T, b1c, W2T, b2c, W3T, b3c)


def _byte_alias(table):
    padded = jnp.concatenate([table, table[:64]], axis=0)
    return padded.reshape(7813, 128, D).transpose(0, 2, 1).reshape(-1)


def kernel(users, tracks, user_table, track_table, W1, b1, W2, b2, W3, b3):
    xT = _sc_gather(users, tracks,
                    _byte_alias(user_table), _byte_alias(track_table))
    oT = _mlp(xT, W1.T, b1.reshape(64, 1), W2.T, b2.reshape(32, 1),
              W3.T, b3.reshape(1, 1))
    return oT.reshape(BATCH, 1)


# speedup vs baseline: 1.0591x; 1.0591x over previous
import jax
import jax.numpy as jnp
from jax import lax
from jax.experimental import pallas as pl
from jax.experimental.pallas import tpu as pltpu
from jax.experimental.pallas import tpu_sc as plsc

BATCH = 16384
D = 8
NUM_CORES = 2
NUM_SUBCORES = 16
NW = NUM_CORES * NUM_SUBCORES
BPW = BATCH // NW
L = 16
NCHUNK = BPW // L


def _sc_gather_body(users_hbm, tracks_hbm, utab_hbm, ttab_hbm,
                    x_out, uidx_v, tidx_v, uaddr_v, taddr_v,
                    urows_v, trows_v, sem):
    c = lax.axis_index("c")
    s = lax.axis_index("s")
    wid = s * NUM_CORES + c
    base = wid * BPW

    cu0 = pltpu.async_copy(users_hbm.at[pl.ds(base, BPW)], uidx_v, sem)
    ct0 = pltpu.async_copy(tracks_hbm.at[pl.ds(base, BPW)], tidx_v, sem)
    cu0.wait()
    ct0.wait()

    def build_addrs(idx_v, addr_v):
        for ch in range(NCHUNK):
            v = idx_v[pl.ds(ch * L, L)]
            a0 = (lax.shift_right_logical(v, 7) * 1024) + (v & 127)
            for k in range(D):
                addr_v[pl.ds(k * BPW + ch * L, L)] = a0 + k * 128

    build_addrs(uidx_v, uaddr_v)
    cu = pltpu.async_copy(utab_hbm.at[uaddr_v], urows_v, sem)
    build_addrs(tidx_v, taddr_v)
    ct = pltpu.async_copy(ttab_hbm.at[taddr_v], trows_v, sem)
    cu.wait()
    ct.wait()
    for k in range(D):
        pltpu.sync_copy(urows_v.at[pl.ds(k * BPW, BPW)],
                        x_out.at[k, pl.ds(base, BPW)])
        pltpu.sync_copy(trows_v.at[pl.ds(k * BPW, BPW)],
                        x_out.at[D + k, pl.ds(base, BPW)])


_sc_gather = pl.kernel(
    _sc_gather_body,
    out_type=jax.ShapeDtypeStruct((2 * D, BATCH), jnp.float32),
    mesh=plsc.VectorSubcoreMesh(core_axis_name="c", subcore_axis_name="s"),
    scratch_types=[
        pltpu.VMEM((BPW,), jnp.int32),
        pltpu.VMEM((BPW,), jnp.int32),
        pltpu.VMEM((D * BPW,), jnp.int32),
        pltpu.VMEM((D * BPW,), jnp.int32),
        pltpu.VMEM((D * BPW,), jnp.float32),
        pltpu.VMEM((D * BPW,), jnp.float32),
        pltpu.SemaphoreType.DMA,
    ],
    compiler_params=pltpu.CompilerParams(use_tc_tiling_on_sc=False),
)


def _mlp_body(x_ref, w1t_ref, b1_ref, w2t_ref, b2_ref,
              w3t_ref, b3_ref, o_ref):
    h = jnp.dot(w1t_ref[...], x_ref[...], preferred_element_type=jnp.float32)
    h = jnp.maximum(h + b1_ref[...], 0.0)
    h = jnp.maximum(
        jnp.dot(w2t_ref[...], h, preferred_element_type=jnp.float32)
        + b2_ref[...], 0.0)
    o = jnp.dot(w3t_ref[...], h, preferred_element_type=jnp.float32) + b3_ref[...]
    o_ref[...] = jax.nn.sigmoid(o)


def _mlp(xT, W1T, b1c, W2T, b2c, W3T, b3c, block=2048):
    grid = BATCH // block
    full = lambda shape: pl.BlockSpec(shape, lambda i: (0, 0))
    return pl.pallas_call(
        _mlp_body,
        grid=(grid,),
        in_specs=[
            pl.BlockSpec((2 * D, block), lambda i: (0, i)),
            full((64, 2 * D)),
            full((64, 1)),
            full((32, 64)),
            full((32, 1)),
            full((1, 32)),
            full((1, 1)),
        ],
        out_specs=pl.BlockSpec((1, block), lambda i: (0, i)),
        out_shape=jax.ShapeDtypeStruct((1, BATCH), jnp.float32),
    )(xT, W1T, b1c, W2T, b2c, W3T, b3c)


def _byte_alias(table):
    padded = jnp.pad(table, ((0, 64), (0, 0)))
    return padded.reshape(7813, 128, D).transpose(0, 2, 1).reshape(-1)


def kernel(users, tracks, user_table, track_table, W1, b1, W2, b2, W3, b3):
    xT = _sc_gather(users, tracks,
                    _byte_alias(user_table), _byte_alias(track_table))
    oT = _mlp(xT, W1.T, b1.reshape(64, 1), W2.T, b2.reshape(32, 1),
              W3.T, b3.reshape(1, 1))
    return oT.reshape(BATCH, 1)


# speedup vs baseline: 1.0936x; 1.0326x over previous
import jax
import jax.numpy as jnp
from jax import lax
from jax.experimental import pallas as pl
from jax.experimental.pallas import tpu as pltpu
from jax.experimental.pallas import tpu_sc as plsc

BATCH = 16384
D = 8
NUM_CORES = 2
NUM_SUBCORES = 16
NW = NUM_CORES * NUM_SUBCORES
BPW = BATCH // NW
L = 16
NCHUNK = BPW // L


def _sc_gather_body(users_hbm, tracks_hbm, utab_hbm, ttab_hbm,
                    x_out, uidx_v, tidx_v, uaddr_v, taddr_v,
                    urows_v, trows_v, sem):
    c = lax.axis_index("c")
    s = lax.axis_index("s")
    wid = s * NUM_CORES + c
    base = wid * BPW

    cu0 = pltpu.async_copy(users_hbm.at[pl.ds(base, BPW)], uidx_v, sem)
    ct0 = pltpu.async_copy(tracks_hbm.at[pl.ds(base, BPW)], tidx_v, sem)
    cu0.wait()
    ct0.wait()

    def build_addrs(idx_v, addr_v):
        for ch in range(NCHUNK):
            v = idx_v[pl.ds(ch * L, L)]
            a0 = (lax.shift_right_logical(v, 7) * 1024) + (v & 127)
            for k in range(D):
                addr_v[pl.ds(k * BPW + ch * L, L)] = a0 + k * 128

    build_addrs(uidx_v, uaddr_v)
    cu = pltpu.async_copy(utab_hbm.at[uaddr_v], urows_v, sem)
    build_addrs(tidx_v, taddr_v)
    ct = pltpu.async_copy(ttab_hbm.at[taddr_v], trows_v, sem)
    cu.wait()
    ct.wait()
    for k in range(D):
        pltpu.sync_copy(urows_v.at[pl.ds(k * BPW, BPW)],
                        x_out.at[k, pl.ds(base, BPW)])
        pltpu.sync_copy(trows_v.at[pl.ds(k * BPW, BPW)],
                        x_out.at[D + k, pl.ds(base, BPW)])


_sc_gather = pl.kernel(
    _sc_gather_body,
    out_type=jax.ShapeDtypeStruct((2 * D, BATCH), jnp.float32),
    mesh=plsc.VectorSubcoreMesh(core_axis_name="c", subcore_axis_name="s"),
    scratch_types=[
        pltpu.VMEM((BPW,), jnp.int32),
        pltpu.VMEM((BPW,), jnp.int32),
        pltpu.VMEM((D * BPW,), jnp.int32),
        pltpu.VMEM((D * BPW,), jnp.int32),
        pltpu.VMEM((D * BPW,), jnp.float32),
        pltpu.VMEM((D * BPW,), jnp.float32),
        pltpu.SemaphoreType.DMA,
    ],
    compiler_params=pltpu.CompilerParams(use_tc_tiling_on_sc=False),
)


def _mlp_body(x_ref, w1t_ref, b1_ref, w2t_ref, b2_ref,
              w3t_ref, b3_ref, o_ref):
    h = jnp.dot(w1t_ref[...], x_ref[...], preferred_element_type=jnp.float32)
    h = jnp.maximum(h + b1_ref[...], 0.0)
    h = jnp.maximum(
        jnp.dot(w2t_ref[...], h, preferred_element_type=jnp.float32)
        + b2_ref[...], 0.0)
    o = jnp.dot(w3t_ref[...], h, preferred_element_type=jnp.float32) + b3_ref[...]
    o_ref[...] = jax.nn.sigmoid(o)


def _mlp(xT, W1T, b1c, W2T, b2c, W3T, b3c, block=4096):
    grid = BATCH // block
    full = lambda shape: pl.BlockSpec(shape, lambda i: (0, 0))
    return pl.pallas_call(
        _mlp_body,
        grid=(grid,),
        in_specs=[
            pl.BlockSpec((2 * D, block), lambda i: (0, i)),
            full((64, 2 * D)),
            full((64, 1)),
            full((32, 64)),
            full((32, 1)),
            full((1, 32)),
            full((1, 1)),
        ],
        out_specs=pl.BlockSpec((1, block), lambda i: (0, i)),
        out_shape=jax.ShapeDtypeStruct((1, BATCH), jnp.float32),
    )(xT, W1T, b1c, W2T, b2c, W3T, b3c)


def _byte_alias(table):
    padded = jnp.pad(table, ((0, 64), (0, 0)))
    return padded.reshape(7813, 128, D).transpose(0, 2, 1).reshape(-1)


def kernel(users, tracks, user_table, track_table, W1, b1, W2, b2, W3, b3):
    xT = _sc_gather(users, tracks,
                    _byte_alias(user_table), _byte_alias(track_table))
    oT = _mlp(xT, W1.T, b1.reshape(64, 1), W2.T, b2.reshape(32, 1),
              W3.T, b3.reshape(1, 1))
    return oT.reshape(BATCH, 1)


# speedup vs baseline: 1.1083x; 1.0135x over previous
import jax
import jax.numpy as jnp
from jax import lax
from jax.experimental import pallas as pl
from jax.experimental.pallas import tpu as pltpu
from jax.experimental.pallas import tpu_sc as plsc

BATCH = 16384
D = 8
NUM_CORES = 2
NUM_SUBCORES = 16
NW = NUM_CORES * NUM_SUBCORES
BPW = BATCH // NW
L = 16
NCHUNK = BPW // L


def _sc_gather_body(users_hbm, tracks_hbm, utab_hbm, ttab_hbm,
                    x_out, uidx_v, tidx_v, uaddr_v, taddr_v,
                    urows_v, trows_v, sem):
    c = lax.axis_index("c")
    s = lax.axis_index("s")
    wid = s * NUM_CORES + c
    base = wid * BPW

    cu0 = pltpu.async_copy(users_hbm.at[pl.ds(base, BPW)], uidx_v, sem)
    ct0 = pltpu.async_copy(tracks_hbm.at[pl.ds(base, BPW)], tidx_v, sem)
    cu0.wait()
    ct0.wait()

    def build_addrs(idx_v, addr_v):
        for ch in range(NCHUNK):
            v = idx_v[pl.ds(ch * L, L)]
            a0 = (lax.shift_right_logical(v, 7) * 1024) + (v & 127)
            for k in range(D):
                addr_v[pl.ds(k * BPW + ch * L, L)] = a0 + k * 128

    build_addrs(uidx_v, uaddr_v)
    cu = pltpu.async_copy(utab_hbm.at[uaddr_v], urows_v, sem)
    build_addrs(tidx_v, taddr_v)
    ct = pltpu.async_copy(ttab_hbm.at[taddr_v], trows_v, sem)
    cu.wait()
    ct.wait()
    for k in range(D):
        pltpu.sync_copy(urows_v.at[pl.ds(k * BPW, BPW)],
                        x_out.at[k, pl.ds(base, BPW)])
        pltpu.sync_copy(trows_v.at[pl.ds(k * BPW, BPW)],
                        x_out.at[D + k, pl.ds(base, BPW)])


_sc_gather = pl.kernel(
    _sc_gather_body,
    out_type=jax.ShapeDtypeStruct((2 * D, BATCH), jnp.float32),
    mesh=plsc.VectorSubcoreMesh(core_axis_name="c", subcore_axis_name="s"),
    scratch_types=[
        pltpu.VMEM((BPW,), jnp.int32),
        pltpu.VMEM((BPW,), jnp.int32),
        pltpu.VMEM((D * BPW,), jnp.int32),
        pltpu.VMEM((D * BPW,), jnp.int32),
        pltpu.VMEM((D * BPW,), jnp.float32),
        pltpu.VMEM((D * BPW,), jnp.float32),
        pltpu.SemaphoreType.DMA,
    ],
    compiler_params=pltpu.CompilerParams(use_tc_tiling_on_sc=False),
)


def _mlp_body(x_ref, w1t_ref, b1_ref, w2t_ref, b2_ref,
              w3t_ref, b3_ref, o_ref):
    h = jnp.dot(w1t_ref[...], x_ref[...], preferred_element_type=jnp.float32)
    h = jnp.maximum(h + b1_ref[...], 0.0)
    h = jnp.maximum(
        jnp.dot(w2t_ref[...], h, preferred_element_type=jnp.float32)
        + b2_ref[...], 0.0)
    o = jnp.dot(w3t_ref[...], h, preferred_element_type=jnp.float32) + b3_ref[...]
    o_ref[...] = jax.nn.sigmoid(o)


def _mlp(xT, W1T, b1c, W2T, b2c, W3T, b3c, block=8192):
    grid = BATCH // block
    full = lambda shape: pl.BlockSpec(shape, lambda i: (0, 0))
    return pl.pallas_call(
        _mlp_body,
        grid=(grid,),
        in_specs=[
            pl.BlockSpec((2 * D, block), lambda i: (0, i)),
            full((64, 2 * D)),
            full((64, 1)),
            full((32, 64)),
            full((32, 1)),
            full((1, 32)),
            full((1, 1)),
        ],
        out_specs=pl.BlockSpec((1, block), lambda i: (0, i)),
        out_shape=jax.ShapeDtypeStruct((1, BATCH), jnp.float32),
    )(xT, W1T, b1c, W2T, b2c, W3T, b3c)


def _byte_alias(table):
    padded = jnp.pad(table, ((0, 64), (0, 0)))
    return padded.reshape(7813, 128, D).transpose(0, 2, 1).reshape(-1)


def kernel(users, tracks, user_table, track_table, W1, b1, W2, b2, W3, b3):
    xT = _sc_gather(users, tracks,
                    _byte_alias(user_table), _byte_alias(track_table))
    oT = _mlp(xT, W1.T, b1.reshape(64, 1), W2.T, b2.reshape(32, 1),
              W3.T, b3.reshape(1, 1))
    return oT.reshape(BATCH, 1)


# speedup vs baseline: 1.1108x; 1.0022x over previous
import jax
import jax.numpy as jnp
from jax import lax
from jax.experimental import pallas as pl
from jax.experimental.pallas import tpu as pltpu
from jax.experimental.pallas import tpu_sc as plsc

BATCH = 16384
D = 8
NUM_CORES = 2
NUM_SUBCORES = 16
NW = NUM_CORES * NUM_SUBCORES
BPW = BATCH // NW
L = 16
NCHUNK = BPW // L


def _sc_gather_body(users_hbm, tracks_hbm, utab_hbm, ttab_hbm,
                    x_out, uidx_v, tidx_v, uaddr_v, taddr_v,
                    urows_v, trows_v, sem):
    c = lax.axis_index("c")
    s = lax.axis_index("s")
    wid = s * NUM_CORES + c
    base = wid * BPW

    cu0 = pltpu.async_copy(users_hbm.at[pl.ds(base, BPW)], uidx_v, sem)
    ct0 = pltpu.async_copy(tracks_hbm.at[pl.ds(base, BPW)], tidx_v, sem)
    cu0.wait()
    ct0.wait()

    def build_addrs(idx_v, addr_v):
        for ch in range(NCHUNK):
            v = idx_v[pl.ds(ch * L, L)]
            a0 = (lax.shift_right_logical(v, 7) * 1024) + (v & 127)
            for k in range(D):
                addr_v[pl.ds(k * BPW + ch * L, L)] = a0 + k * 128

    build_addrs(uidx_v, uaddr_v)
    cu = pltpu.async_copy(utab_hbm.at[uaddr_v], urows_v, sem)
    build_addrs(tidx_v, taddr_v)
    ct = pltpu.async_copy(ttab_hbm.at[taddr_v], trows_v, sem)
    cu.wait()
    ct.wait()
    for k in range(D):
        pltpu.sync_copy(urows_v.at[pl.ds(k * BPW, BPW)],
                        x_out.at[k, pl.ds(base, BPW)])
        pltpu.sync_copy(trows_v.at[pl.ds(k * BPW, BPW)],
                        x_out.at[D + k, pl.ds(base, BPW)])


_sc_gather = pl.kernel(
    _sc_gather_body,
    out_type=jax.ShapeDtypeStruct((2 * D, BATCH), jnp.float32),
    mesh=plsc.VectorSubcoreMesh(core_axis_name="c", subcore_axis_name="s"),
    scratch_types=[
        pltpu.VMEM((BPW,), jnp.int32),
        pltpu.VMEM((BPW,), jnp.int32),
        pltpu.VMEM((D * BPW,), jnp.int32),
        pltpu.VMEM((D * BPW,), jnp.int32),
        pltpu.VMEM((D * BPW,), jnp.float32),
        pltpu.VMEM((D * BPW,), jnp.float32),
        pltpu.SemaphoreType.DMA,
    ],
    compiler_params=pltpu.CompilerParams(use_tc_tiling_on_sc=False),
)


def _mlp_body(x_ref, w1t_ref, b1_ref, w2t_ref, b2_ref,
              w3t_ref, b3_ref, o_ref):
    h = jnp.dot(w1t_ref[...], x_ref[...], preferred_element_type=jnp.float32)
    h = jnp.maximum(h + b1_ref[...], 0.0)
    h = jnp.maximum(
        jnp.dot(w2t_ref[...], h, preferred_element_type=jnp.float32)
        + b2_ref[...], 0.0)
    o = jnp.dot(w3t_ref[...], h, preferred_element_type=jnp.float32) + b3_ref[...]
    o_ref[...] = jax.nn.sigmoid(o)


def _mlp(xT, W1T, b1c, W2T, b2c, W3T, b3c, block=16384):
    grid = BATCH // block
    full = lambda shape: pl.BlockSpec(shape, lambda i: (0, 0))
    return pl.pallas_call(
        _mlp_body,
        grid=(grid,),
        in_specs=[
            pl.BlockSpec((2 * D, block), lambda i: (0, i)),
            full((64, 2 * D)),
            full((64, 1)),
            full((32, 64)),
            full((32, 1)),
            full((1, 32)),
            full((1, 1)),
        ],
        out_specs=pl.BlockSpec((1, block), lambda i: (0, i)),
        out_shape=jax.ShapeDtypeStruct((1, BATCH), jnp.float32),
    )(xT, W1T, b1c, W2T, b2c, W3T, b3c)


def _byte_alias(table):
    padded = jnp.pad(table, ((0, 64), (0, 0)))
    return padded.reshape(7813, 128, D).transpose(0, 2, 1).reshape(-1)


def kernel(users, tracks, user_table, track_table, W1, b1, W2, b2, W3, b3):
    xT = _sc_gather(users, tracks,
                    _byte_alias(user_table), _byte_alias(track_table))
    oT = _mlp(xT, W1.T, b1.reshape(64, 1), W2.T, b2.reshape(32, 1),
              W3.T, b3.reshape(1, 1))
    return oT.reshape(BATCH, 1)
